# baseline (speedup 1.0000x reference)
import jax
import jax.numpy as jnp
from jax import lax
from jax.experimental import pallas as pl
from jax.experimental.pallas import tpu as pltpu

_N, _E, _D = 4096, 8, 768
_CB = 256
_NCH = _N // _CB
_NBUF = 6
_LOOK = 3


def _body(x_hbm, act_hbm, gate_hbm, disp_hbm,
          x_v, gate_v, act_b, disp_b, in_sems, out_sems, out_sems2, gsem):
    pltpu.make_async_copy(x_hbm, x_v, gsem).start()
    pltpu.make_async_copy(x_hbm, x_v, gsem).wait()
    t = x_v[...] * 2.0
    t2 = t * t
    gate_v[...] = 1.0 / (1.0 + t2 * t2)
    pltpu.make_async_copy(gate_v, gate_hbm, gsem).start()

    def act_in(c):
        return pltpu.make_async_copy(
            act_hbm.at[pl.ds(c * _CB, _CB)],
            act_b.at[pl.ds(c * _CB, _CB)], in_sems.at[c])

    _H = _CB // 2

    def disp_out_h(c, slot, h, sems):
        return pltpu.make_async_copy(
            disp_b.at[slot, pl.ds(h * _H, _H)],
            disp_hbm.at[pl.ds(c * _CB + h * _H, _H)], sems.at[slot])

    for c in range(_NCH):
        act_in(c).start()

    act_in(0).wait()
    _H8 = _CB // 8
    for h8 in range(8):
        sems = out_sems if h8 % 2 == 0 else out_sems2
        g0 = gate_v[pl.ds(h8 * _H8, _H8), :]
        gm0 = jnp.where(g0 >= 0.5, g0, 0.0)
        a0 = act_b[pl.ds(h8 * _H8, _H8)]
        disp_b[0, pl.ds(h8 * _H8, _H8)] = gm0[:, :, None] * a0[:, None, :]
        pltpu.make_async_copy(
            disp_b.at[0, pl.ds(h8 * _H8, _H8)],
            disp_hbm.at[pl.ds(h8 * _H8, _H8)], sems.at[0]).start()

    def step(c, carry):
        slot = lax.rem(c, _NBUF)

        act_in(c).wait()

        @pl.when(c >= _NBUF)
        def _():
            disp_out_h(c - _NBUF, slot, 0, out_sems).wait()
            disp_out_h(c - _NBUF, slot, 1, out_sems2).wait()

        for h, sems in ((0, out_sems), (1, out_sems2)):
            gate = gate_v[pl.ds(c * _CB + h * _H, _H), :]
            gm = jnp.where(gate >= 0.5, gate, 0.0)
            a = act_b[pl.ds(c * _CB + h * _H, _H)]
            disp_b[slot, pl.ds(h * _H, _H)] = gm[:, :, None] * a[:, None, :]
            disp_out_h(c, slot, h, sems).start()
        return carry

    lax.fori_loop(1, _NCH, step, 0, unroll=False)

    for k in range(_NCH - _NBUF, _NCH):
        disp_out_h(k, k % _NBUF, 0, out_sems).wait()
        disp_out_h(k, k % _NBUF, 1, out_sems2).wait()
    pltpu.make_async_copy(gate_v, gate_hbm, gsem).wait()


def kernel(x, act, batch_inds):
    gate, disp = pl.pallas_call(
        _body,
        in_specs=[
            pl.BlockSpec(memory_space=pl.ANY),
            pl.BlockSpec(memory_space=pl.ANY),
        ],
        out_specs=[
            pl.BlockSpec(memory_space=pl.ANY),
            pl.BlockSpec(memory_space=pl.ANY),
        ],
        out_shape=[
            jax.ShapeDtypeStruct((_N, _E), jnp.float32),
            jax.ShapeDtypeStruct((_N, _E, _D), jnp.float32),
        ],
        scratch_shapes=[
            pltpu.VMEM((_N, _E), jnp.float32),
            pltpu.VMEM((_N, _E), jnp.float32),
            pltpu.VMEM((_N, _D), jnp.float32),
            pltpu.VMEM((_NBUF, _CB, _E, _D), jnp.float32),
            pltpu.SemaphoreType.DMA((_NCH,)),
            pltpu.SemaphoreType.DMA((_NBUF,)),
            pltpu.SemaphoreType.DMA((_NBUF,)),
            pltpu.SemaphoreType.DMA,
        ],
    )(x, act)
    return gate, disp
